# Initial kernel scaffold; baseline (speedup 1.0000x reference)
import jax
import jax.numpy as jnp
from jax.experimental import pallas as pl


def kernel(x, edge_index, edge_type, batch, color_emb, pos_emb, W1, root1, b1, W2, root2, b2, lin_W, lin_b):
    raise NotImplementedError("write your pallas kernel here")


# speedup vs baseline: 1.3089x; 1.3089x over previous
import functools

import jax
import jax.numpy as jnp
from jax import lax
from jax.experimental import pallas as pl
from jax.experimental.pallas import tpu as pltpu
from jax.experimental.pallas import tpu_sc as plsc

N_NODES = 10000
N_EDGES = 320000
N_COLOR = 16
MAX_POS = 1024
EMB = 128
HID = 256
N_CLASS = 10
N_REL = 3
N_GRAPHS = 64

HALF = HID // 2
NC = 2
NS = 16
RB = 1000
GRID = N_NODES // RB
EPT = N_EDGES // NS
SCH = 400
ACH = 80
NCH_A = EPT // ACH
ACC_PAD = 10240
ROWS_PT = ACC_PAD // NS
CNT_PAD = 30720
ZPT = CNT_PAD // NS
ZSTEP = 128

_f32 = jnp.float32
_i32 = jnp.int32


def _mm(a, b):
  return lax.dot_general(a, b, (((1,), (0,)), ((), ())),
                         precision=lax.Precision.HIGHEST,
                         preferred_element_type=_f32)


def _t1_body(c_ref, p_ref, ce_ref, pe_ref, root_ref, b_ref, w_ref,
             r1_ref, th_ref):
  c = c_ref[:, 0]
  p = p_ref[:, 0]
  ohc = (c[:, None] == lax.broadcasted_iota(_i32, (RB, N_COLOR), 1)
         ).astype(_f32)
  ohp = (p[:, None] == lax.broadcasted_iota(_i32, (RB, MAX_POS), 1)
         ).astype(_f32)
  h0 = _mm(ohc, ce_ref[...]) + _mm(ohp, pe_ref[...])
  r1_ref[...] = _mm(h0, root_ref[...]) + b_ref[...]
  for ch in range(NC):
    for r in range(N_REL):
      th_ref[ch, r] = _mm(h0, w_ref[ch, r])


_t1_call = pl.pallas_call(
    _t1_body,
    grid=(GRID,),
    in_specs=[
        pl.BlockSpec((RB, 1), lambda i: (i, 0)),
        pl.BlockSpec((RB, 1), lambda i: (i, 0)),
        pl.BlockSpec((N_COLOR, EMB), lambda i: (0, 0)),
        pl.BlockSpec((MAX_POS, EMB), lambda i: (0, 0)),
        pl.BlockSpec((EMB, HID), lambda i: (0, 0)),
        pl.BlockSpec((1, HID), lambda i: (0, 0)),
        pl.BlockSpec((NC, N_REL, EMB, HALF), lambda i: (0, 0, 0, 0)),
    ],
    out_specs=[
        pl.BlockSpec((RB, HID), lambda i: (i, 0)),
        pl.BlockSpec((NC, N_REL, RB, HALF), lambda i: (0, 0, i, 0)),
    ],
    out_shape=[
        jax.ShapeDtypeStruct((N_NODES, HID), _f32),
        jax.ShapeDtypeStruct((NC, N_REL, N_NODES, HALF), _f32),
    ],
)


def _t2_body(rin_ref, acc_ref, root_ref, b_ref, w_ref, r2_ref, th_ref):
  h = jnp.maximum(
      rin_ref[...] + jnp.concatenate([acc_ref[0], acc_ref[1]], axis=-1), 0.0)
  r2_ref[...] = _mm(h, root_ref[...]) + b_ref[...]
  for ch in range(NC):
    for r in range(N_REL):
      th_ref[ch, r] = _mm(h, w_ref[ch, r])


_t2_call = pl.pallas_call(
    _t2_body,
    grid=(GRID,),
    in_specs=[
        pl.BlockSpec((RB, HID), lambda i: (i, 0)),
        pl.BlockSpec((NC, RB, HALF), lambda i: (0, i, 0)),
        pl.BlockSpec((HID, HID), lambda i: (0, 0)),
        pl.BlockSpec((1, HID), lambda i: (0, 0)),
        pl.BlockSpec((NC, N_REL, HID, HALF), lambda i: (0, 0, 0, 0)),
    ],
    out_specs=[
        pl.BlockSpec((RB, HID), lambda i: (i, 0)),
        pl.BlockSpec((NC, N_REL, RB, HALF), lambda i: (0, 0, i, 0)),
    ],
    out_shape=[
        jax.ShapeDtypeStruct((N_NODES, HID), _f32),
        jax.ShapeDtypeStruct((NC, N_REL, N_NODES, HALF), _f32),
    ],
)


def _t3_body(rin_ref, acc_ref, b_ref, lw_ref, lb_ref, out_ref,
             pool_ref, cnt_ref):
  i = pl.program_id(0)

  @pl.when(i == 0)
  def _():
    pool_ref[...] = jnp.zeros_like(pool_ref)
    cnt_ref[...] = jnp.zeros_like(cnt_ref)

  h = jnp.maximum(
      rin_ref[...] + jnp.concatenate([acc_ref[0], acc_ref[1]], axis=-1), 0.0)
  bid = b_ref[:, 0]
  oh = (bid[:, None] == lax.broadcasted_iota(_i32, (RB, N_GRAPHS), 1)
        ).astype(_f32)
  tdot = functools.partial(lax.dot_general,
                           dimension_numbers=(((0,), (0,)), ((), ())),
                           precision=lax.Precision.HIGHEST,
                           preferred_element_type=_f32)
  pool_ref[...] += tdot(oh, h)
  cnt_ref[...] += tdot(oh, jnp.ones((RB, 128), _f32))

  @pl.when(i == GRID - 1)
  def _():
    pooled = pool_ref[...] / jnp.maximum(cnt_ref[:, :1], 1.0)
    out_ref[...] = _mm(pooled, lw_ref[...]) + lb_ref[...]


_t3_call = pl.pallas_call(
    _t3_body,
    grid=(GRID,),
    in_specs=[
        pl.BlockSpec((RB, HID), lambda i: (i, 0)),
        pl.BlockSpec((NC, RB, HALF), lambda i: (0, i, 0)),
        pl.BlockSpec((RB, 1), lambda i: (i, 0)),
        pl.BlockSpec((HID, 128), lambda i: (0, 0)),
        pl.BlockSpec((1, 128), lambda i: (0, 0)),
    ],
    out_specs=pl.BlockSpec((N_GRAPHS, 128), lambda i: (0, 0)),
    out_shape=jax.ShapeDtypeStruct((N_GRAPHS, 128), _f32),
    scratch_shapes=[
        pltpu.VMEM((N_GRAPHS, HID), _f32),
        pltpu.VMEM((N_GRAPHS, 128), _f32),
    ],
)


_sc_mesh = plsc.VectorSubcoreMesh(core_axis_name="c", subcore_axis_name="s")


@functools.partial(
    pl.kernel,
    out_type=jax.ShapeDtypeStruct((N_EDGES,), _f32),
    mesh=_sc_mesh,
    compiler_params=pltpu.CompilerParams(needs_layout_passes=False),
    scratch_types=[
        pltpu.VMEM_SHARED((CNT_PAD,), _f32),
        pltpu.VMEM((SCH,), _i32),
        pltpu.VMEM((SCH,), _f32),
        pltpu.VMEM((SCH,), _f32),
        pltpu.VMEM((CNT_PAD,), _f32),
    ],
)
def _sc_scales(cidx_hbm, s_hbm, cnt_sh, idx_v, ones_v, s_v, cnt_v):
  cid = lax.axis_index("c")
  sid = lax.axis_index("s")

  def fill(i, carry):
    ones_v[pl.ds(i * 16, 16)] = jnp.full((16,), 1.0, _f32)
    s_v[pl.ds(i * 16, 16)] = jnp.zeros((16,), _f32)
    return carry
  lax.fori_loop(0, SCH // 16, fill, 0)

  def zero(i, carry):
    pltpu.sync_copy(s_v.at[pl.ds(0, ZSTEP)],
                    cnt_sh.at[pl.ds(sid * ZPT + i * ZSTEP, ZSTEP)])
    return carry
  lax.fori_loop(0, ZPT // ZSTEP, zero, 0)
  plsc.subcore_barrier()

  def chunk_a(i, carry):
    base = sid * EPT + i * SCH
    pltpu.sync_copy(cidx_hbm.at[pl.ds(base, SCH)], idx_v)
    pltpu.sync_copy(ones_v, cnt_sh.at[idx_v], add=True)
    return carry
  lax.fori_loop(0, EPT // SCH, chunk_a, 0)
  plsc.subcore_barrier()

  pltpu.sync_copy(cnt_sh, cnt_v)
  ebase = cid * (N_EDGES // NC) + sid * (EPT // NC)

  def chunk_b(i, carry):
    b = ebase + i * SCH
    pltpu.sync_copy(cidx_hbm.at[pl.ds(b, SCH)], idx_v)

    def grp(j, c2):
      iv = idx_v[pl.ds(j * 16, 16)]
      cv = plsc.load_gather(cnt_v, [iv])
      s_v[pl.ds(j * 16, 16)] = 1.0 / jnp.maximum(cv, 1.0)
      return c2
    lax.fori_loop(0, SCH // 16, grp, 0)
    pltpu.sync_copy(s_v, s_hbm.at[pl.ds(b, SCH)])
    return carry
  lax.fori_loop(0, EPT // NC // SCH, chunk_b, 0)


@functools.partial(
    pl.kernel,
    out_type=jax.ShapeDtypeStruct((NC * N_NODES, HALF), _f32),
    mesh=_sc_mesh,
    compiler_params=pltpu.CompilerParams(needs_layout_passes=False),
    scratch_types=[
        pltpu.VMEM_SHARED((ACC_PAD, HALF), _f32),
        pltpu.VMEM((ACH,), _i32),
        pltpu.VMEM((ACH,), _i32),
        pltpu.VMEM((ACH,), _f32),
        pltpu.VMEM((ACH, HALF), _f32),
        pltpu.VMEM((16, HALF), _f32),
        pltpu.SemaphoreType.DMA,
    ],
)
def _sc_agg(th_hbm, g_hbm, dst_hbm, s_hbm, out_hbm,
            acc_sh, g_v, d_v, s_v, rows_v, z_v, sem):
  cid = lax.axis_index("c")
  sid = lax.axis_index("s")

  def zfill(i, carry):
    r = i // (HALF // 16)
    k = i % (HALF // 16)
    z_v[r, pl.ds(k * 16, 16)] = jnp.zeros((16,), _f32)
    return carry
  lax.fori_loop(0, 16 * (HALF // 16), zfill, 0)

  rbase = sid * ROWS_PT

  def zero(i, carry):
    pltpu.sync_copy(z_v, acc_sh.at[pl.ds(rbase + i * 16, 16)])
    return carry
  lax.fori_loop(0, ROWS_PT // 16, zero, 0)
  plsc.subcore_barrier()

  off = cid * (N_REL * N_NODES)

  def chunk(i, carry):
    b = sid * EPT + i * ACH
    pltpu.sync_copy(g_hbm.at[pl.ds(b, ACH)], g_v)
    pltpu.sync_copy(dst_hbm.at[pl.ds(b, ACH)], d_v)
    pltpu.sync_copy(s_hbm.at[pl.ds(b, ACH)], s_v)

    def adj(j, c2):
      g_v[pl.ds(j * 16, 16)] = g_v[pl.ds(j * 16, 16)] + off
      return c2
    lax.fori_loop(0, ACH // 16, adj, 0)

    pltpu.async_copy(th_hbm.at[g_v], rows_v, sem).wait()

    def scale(j, c2):
      sv = s_v[pl.ds(j * 16, 16)]
      rowi = j * 16 + lax.iota(_i32, 16)

      def feat(k, c3):
        col = jnp.zeros((16,), _i32) + k
        v = plsc.load_gather(rows_v, [rowi, col])
        plsc.store_scatter(rows_v, [rowi, col], v * sv)
        return c3
      lax.fori_loop(0, HALF, feat, 0)
      return c2
    lax.fori_loop(0, ACH // 16, scale, 0)

    pltpu.sync_copy(rows_v, acc_sh.at[d_v], add=True)
    return carry
  lax.fori_loop(0, NCH_A, chunk, 0)
  plsc.subcore_barrier()

  @pl.when(sid < NS - 1)
  def _():
    pltpu.sync_copy(acc_sh.at[pl.ds(rbase, ROWS_PT)],
                    out_hbm.at[pl.ds(cid * N_NODES + rbase, ROWS_PT)])

  @pl.when(sid == NS - 1)
  def _():
    pltpu.sync_copy(acc_sh.at[pl.ds(rbase, N_NODES - (NS - 1) * ROWS_PT)],
                    out_hbm.at[pl.ds(cid * N_NODES + rbase,
                                     N_NODES - (NS - 1) * ROWS_PT)])


def kernel(x, edge_index, edge_type, batch, color_emb, pos_emb,
           W1, root1, b1, W2, root2, b2, lin_W, lin_b):
  xi = x.astype(_i32)
  c2 = xi[:, 1:2]
  p2 = xi[:, 2:3]
  src = edge_index[0].astype(_i32)
  dst = edge_index[1].astype(_i32)
  rel = edge_type.astype(_i32)
  g = rel * N_NODES + src
  cidx = rel * N_NODES + dst
  batch2 = batch.astype(_i32)[:, None]

  W1h = W1.reshape(N_REL, EMB, NC, HALF).transpose(2, 0, 1, 3)
  W2h = W2.reshape(N_REL, HID, NC, HALF).transpose(2, 0, 1, 3)
  b1r = b1[None, :]
  b2r = b2[None, :]
  lwp = jnp.zeros((HID, 128), _f32).at[:, :N_CLASS].set(lin_W)
  lbp = jnp.zeros((1, 128), _f32).at[:, :N_CLASS].set(lin_b[None, :])

  s_edge = _sc_scales(cidx)

  r1, t1h = _t1_call(c2, p2, color_emb, pos_emb, root1, b1r, W1h)
  acc1 = _sc_agg(t1h.reshape(NC * N_REL * N_NODES, HALF), g, dst, s_edge)
  acc1 = acc1.reshape(NC, N_NODES, HALF)

  r2, t2h = _t2_call(r1, acc1, root2, b2r, W2h)
  acc2 = _sc_agg(t2h.reshape(NC * N_REL * N_NODES, HALF), g, dst, s_edge)
  acc2 = acc2.reshape(NC, N_NODES, HALF)

  out = _t3_call(r2, acc2, batch2, lwp, lbp)
  return out[:, :N_CLASS]


# speedup vs baseline: 9.7795x; 7.4717x over previous
import functools

import jax
import jax.numpy as jnp
from jax import lax
from jax.experimental import pallas as pl
from jax.experimental.pallas import tpu as pltpu
from jax.experimental.pallas import tpu_sc as plsc

N_NODES = 10000
N_EDGES = 320000
N_COLOR = 16
MAX_POS = 1024
EMB = 128
HID = 256
N_CLASS = 10
N_REL = 3
N_GRAPHS = 64

HALF = HID // 2
NC = 2
NS = 16
RB = 1000
GRID = N_NODES // RB
EPT = N_EDGES // NS
SCH = 400
ACH = 80
BLK = 4000
NBLK = EPT // BLK
CPB = BLK // ACH
PAIRS = CPB // 2
CROWS = EPT // ACH
ACC_PAD = 10240
ROWS_PT = ACC_PAD // NS
CNT_PAD = 30720
ZPT = CNT_PAD // NS
ZSTEP = 128

_f32 = jnp.float32
_i32 = jnp.int32


def _mm(a, b):
  return lax.dot_general(a, b, (((1,), (0,)), ((), ())),
                         precision=lax.Precision.HIGHEST,
                         preferred_element_type=_f32)


def _t1_body(c_ref, p_ref, ce_ref, pe_ref, root_ref, b_ref, w_ref,
             r1_ref, th_ref):
  c = c_ref[:, 0]
  p = p_ref[:, 0]
  ohc = (c[:, None] == lax.broadcasted_iota(_i32, (RB, N_COLOR), 1)
         ).astype(_f32)
  ohp = (p[:, None] == lax.broadcasted_iota(_i32, (RB, MAX_POS), 1)
         ).astype(_f32)
  h0 = _mm(ohc, ce_ref[...]) + _mm(ohp, pe_ref[...])
  r1_ref[...] = _mm(h0, root_ref[...]) + b_ref[...]
  for ch in range(NC):
    for r in range(N_REL):
      th_ref[ch, r] = _mm(h0, w_ref[ch, r])


_t1_call = pl.pallas_call(
    _t1_body,
    grid=(GRID,),
    in_specs=[
        pl.BlockSpec((RB, 1), lambda i: (i, 0)),
        pl.BlockSpec((RB, 1), lambda i: (i, 0)),
        pl.BlockSpec((N_COLOR, EMB), lambda i: (0, 0)),
        pl.BlockSpec((MAX_POS, EMB), lambda i: (0, 0)),
        pl.BlockSpec((EMB, HID), lambda i: (0, 0)),
        pl.BlockSpec((1, HID), lambda i: (0, 0)),
        pl.BlockSpec((NC, N_REL, EMB, HALF), lambda i: (0, 0, 0, 0)),
    ],
    out_specs=[
        pl.BlockSpec((RB, HID), lambda i: (i, 0)),
        pl.BlockSpec((NC, N_REL, RB, HALF), lambda i: (0, 0, i, 0)),
    ],
    out_shape=[
        jax.ShapeDtypeStruct((N_NODES, HID), _f32),
        jax.ShapeDtypeStruct((NC, N_REL, N_NODES, HALF), _f32),
    ],
)


def _t2_body(rin_ref, acc_ref, root_ref, b_ref, w_ref, r2_ref, th_ref):
  h = jnp.maximum(
      rin_ref[...] + jnp.concatenate([acc_ref[0], acc_ref[1]], axis=-1), 0.0)
  r2_ref[...] = _mm(h, root_ref[...]) + b_ref[...]
  for ch in range(NC):
    for r in range(N_REL):
      th_ref[ch, r] = _mm(h, w_ref[ch, r])


_t2_call = pl.pallas_call(
    _t2_body,
    grid=(GRID,),
    in_specs=[
        pl.BlockSpec((RB, HID), lambda i: (i, 0)),
        pl.BlockSpec((NC, RB, HALF), lambda i: (0, i, 0)),
        pl.BlockSpec((HID, HID), lambda i: (0, 0)),
        pl.BlockSpec((1, HID), lambda i: (0, 0)),
        pl.BlockSpec((NC, N_REL, HID, HALF), lambda i: (0, 0, 0, 0)),
    ],
    out_specs=[
        pl.BlockSpec((RB, HID), lambda i: (i, 0)),
        pl.BlockSpec((NC, N_REL, RB, HALF), lambda i: (0, 0, i, 0)),
    ],
    out_shape=[
        jax.ShapeDtypeStruct((N_NODES, HID), _f32),
        jax.ShapeDtypeStruct((NC, N_REL, N_NODES, HALF), _f32),
    ],
)


def _t3_body(rin_ref, acc_ref, b_ref, lw_ref, lb_ref, out_ref,
             pool_ref, cnt_ref):
  i = pl.program_id(0)

  @pl.when(i == 0)
  def _():
    pool_ref[...] = jnp.zeros_like(pool_ref)
    cnt_ref[...] = jnp.zeros_like(cnt_ref)

  h = jnp.maximum(
      rin_ref[...] + jnp.concatenate([acc_ref[0], acc_ref[1]], axis=-1), 0.0)
  bid = b_ref[:, 0]
  oh = (bid[:, None] == lax.broadcasted_iota(_i32, (RB, N_GRAPHS), 1)
        ).astype(_f32)
  tdot = functools.partial(lax.dot_general,
                           dimension_numbers=(((0,), (0,)), ((), ())),
                           precision=lax.Precision.HIGHEST,
                           preferred_element_type=_f32)
  pool_ref[...] += tdot(oh, h)
  cnt_ref[...] += tdot(oh, jnp.ones((RB, 128), _f32))

  @pl.when(i == GRID - 1)
  def _():
    pooled = pool_ref[...] / jnp.maximum(cnt_ref[:, :1], 1.0)
    out_ref[...] = _mm(pooled, lw_ref[...]) + lb_ref[...]


_t3_call = pl.pallas_call(
    _t3_body,
    grid=(GRID,),
    in_specs=[
        pl.BlockSpec((RB, HID), lambda i: (i, 0)),
        pl.BlockSpec((NC, RB, HALF), lambda i: (0, i, 0)),
        pl.BlockSpec((RB, 1), lambda i: (i, 0)),
        pl.BlockSpec((HID, 128), lambda i: (0, 0)),
        pl.BlockSpec((1, 128), lambda i: (0, 0)),
    ],
    out_specs=pl.BlockSpec((N_GRAPHS, 128), lambda i: (0, 0)),
    out_shape=jax.ShapeDtypeStruct((N_GRAPHS, 128), _f32),
    scratch_shapes=[
        pltpu.VMEM((N_GRAPHS, HID), _f32),
        pltpu.VMEM((N_GRAPHS, 128), _f32),
    ],
)


_sc_mesh = plsc.VectorSubcoreMesh(core_axis_name="c", subcore_axis_name="s")


@functools.partial(
    pl.kernel,
    out_type=jax.ShapeDtypeStruct((N_EDGES,), _f32),
    mesh=_sc_mesh,
    compiler_params=pltpu.CompilerParams(needs_layout_passes=False),
    scratch_types=[
        pltpu.VMEM_SHARED((CNT_PAD,), _f32),
        pltpu.VMEM((SCH,), _i32),
        pltpu.VMEM((SCH,), _f32),
        pltpu.VMEM((SCH,), _f32),
        pltpu.VMEM((CNT_PAD,), _f32),
    ],
)
def _sc_scales(cidx_hbm, s_hbm, cnt_sh, idx_v, ones_v, s_v, cnt_v):
  cid = lax.axis_index("c")
  sid = lax.axis_index("s")

  def fill(i, carry):
    ones_v[pl.ds(i * 16, 16)] = jnp.full((16,), 1.0, _f32)
    s_v[pl.ds(i * 16, 16)] = jnp.zeros((16,), _f32)
    return carry
  lax.fori_loop(0, SCH // 16, fill, 0)

  def zero(i, carry):
    pltpu.sync_copy(s_v.at[pl.ds(0, ZSTEP)],
                    cnt_sh.at[pl.ds(sid * ZPT + i * ZSTEP, ZSTEP)])
    return carry
  lax.fori_loop(0, ZPT // ZSTEP, zero, 0)
  plsc.subcore_barrier()

  def chunk_a(i, carry):
    base = sid * EPT + i * SCH
    pltpu.sync_copy(cidx_hbm.at[pl.ds(base, SCH)], idx_v)
    pltpu.sync_copy(ones_v, cnt_sh.at[idx_v], add=True)
    return carry
  lax.fori_loop(0, EPT // SCH, chunk_a, 0)
  plsc.subcore_barrier()

  pltpu.sync_copy(cnt_sh, cnt_v)
  ebase = cid * (N_EDGES // NC) + sid * (EPT // NC)

  def chunk_b(i, carry):
    b = ebase + i * SCH
    pltpu.sync_copy(cidx_hbm.at[pl.ds(b, SCH)], idx_v)

    def grp(j, c2):
      iv = idx_v[pl.ds(j * 16, 16)]
      cv = plsc.load_gather(cnt_v, [iv])
      s_v[pl.ds(j * 16, 16)] = 1.0 / jnp.maximum(cv, 1.0)
      return c2
    lax.fori_loop(0, SCH // 16, grp, 0)
    pltpu.sync_copy(s_v, s_hbm.at[pl.ds(b, SCH)])
    return carry
  lax.fori_loop(0, EPT // NC // SCH, chunk_b, 0)


@functools.partial(
    pl.kernel,
    out_type=jax.ShapeDtypeStruct((NC * N_NODES, HALF), _f32),
    mesh=_sc_mesh,
    compiler_params=pltpu.CompilerParams(needs_layout_passes=False),
    scratch_types=[
        pltpu.VMEM_SHARED((ACC_PAD, HALF), _f32),
        pltpu.VMEM((CPB, ACH), _i32),
        pltpu.VMEM((CPB, ACH), _i32),
        pltpu.VMEM((BLK,), _f32),
        pltpu.VMEM((ACH, HALF), _f32),
        pltpu.VMEM((ACH, HALF), _f32),
        pltpu.VMEM((16, HALF), _f32),
        pltpu.SemaphoreType.DMA,
        pltpu.SemaphoreType.DMA,
        pltpu.SemaphoreType.DMA,
    ],
)
def _sc_agg(th_hbm, g_hbm, dst_hbm, s_hbm, out_hbm,
            acc_sh, g2_v, d2_v, s_v, rows_a, rows_b, z_v,
            sem_ga, sem_gb, sem_sa):
  cid = lax.axis_index("c")
  sid = lax.axis_index("s")

  def zfill(i, carry):
    r = i // (HALF // 16)
    k = i % (HALF // 16)
    z_v[r, pl.ds(k * 16, 16)] = jnp.zeros((16,), _f32)
    return carry
  lax.fori_loop(0, 16 * (HALF // 16), zfill, 0)

  rbase = sid * ROWS_PT

  def zero(i, carry):
    pltpu.sync_copy(z_v, acc_sh.at[pl.ds(rbase + i * 16, 16)])
    return carry
  lax.fori_loop(0, ROWS_PT // 16, zero, 0)
  plsc.subcore_barrier()

  off = cid * (N_REL * N_NODES)

  def _scale(rows, c):
    def edge(j, c2):
      sv = plsc.load_gather(s_v, [jnp.full((16,), c * ACH + j, _i32)])
      for k in range(HALF // 16):
        rows[j, pl.ds(k * 16, 16)] = rows[j, pl.ds(k * 16, 16)] * sv
      return c2
    lax.fori_loop(0, ACH, edge, 0)

  def block(b, carry):
    blkid = sid * NBLK + b
    pltpu.sync_copy(g_hbm.at[blkid], g2_v)
    pltpu.sync_copy(dst_hbm.at[blkid], d2_v)
    pltpu.sync_copy(s_hbm.at[pl.ds(sid * EPT + b * BLK, BLK)], s_v)

    def adj(i, c2):
      r = i // (ACH // 16)
      k = i % (ACH // 16)
      g2_v[r, pl.ds(k * 16, 16)] = g2_v[r, pl.ds(k * 16, 16)] + off
      return c2
    lax.fori_loop(0, CPB * (ACH // 16), adj, 0)

    pltpu.async_copy(th_hbm.at[g2_v.at[0]], rows_a, sem_ga)

    def pair(p, c2):
      ca = 2 * p
      cb = 2 * p + 1
      pltpu.async_copy(th_hbm.at[g2_v.at[cb]], rows_b, sem_gb)
      pltpu.make_async_copy(th_hbm.at[g2_v.at[ca]], rows_a, sem_ga).wait()
      _scale(rows_a, ca)
      pltpu.async_copy(rows_a, acc_sh.at[d2_v.at[ca]], sem_sa, add=True)
      pltpu.make_async_copy(th_hbm.at[g2_v.at[cb]], rows_b, sem_gb).wait()
      _scale(rows_b, cb)
      pltpu.make_async_copy(rows_a, acc_sh.at[d2_v.at[ca]], sem_sa).wait()

      @pl.when(p < PAIRS - 1)
      def _():
        pltpu.async_copy(th_hbm.at[g2_v.at[ca + 2]], rows_a, sem_ga)

      pltpu.sync_copy(rows_b, acc_sh.at[d2_v.at[cb]], add=True)
      return c2
    lax.fori_loop(0, PAIRS, pair, 0)
    return carry
  lax.fori_loop(0, NBLK, block, 0)
  plsc.subcore_barrier()

  @pl.when(sid < NS - 1)
  def _():
    pltpu.sync_copy(acc_sh.at[pl.ds(rbase, ROWS_PT)],
                    out_hbm.at[pl.ds(cid * N_NODES + rbase, ROWS_PT)])

  @pl.when(sid == NS - 1)
  def _():
    pltpu.sync_copy(acc_sh.at[pl.ds(rbase, N_NODES - (NS - 1) * ROWS_PT)],
                    out_hbm.at[pl.ds(cid * N_NODES + rbase,
                                     N_NODES - (NS - 1) * ROWS_PT)])


def kernel(x, edge_index, edge_type, batch, color_emb, pos_emb,
           W1, root1, b1, W2, root2, b2, lin_W, lin_b):
  xi = x.astype(_i32)
  c2 = xi[:, 1:2]
  p2 = xi[:, 2:3]
  src = edge_index[0].astype(_i32)
  dst = edge_index[1].astype(_i32)
  rel = edge_type.astype(_i32)
  g2 = (rel * N_NODES + src).reshape(NS * NBLK, CPB, ACH)
  d2 = dst.reshape(NS * NBLK, CPB, ACH)
  cidx = rel * N_NODES + dst
  batch2 = batch.astype(_i32)[:, None]

  W1h = W1.reshape(N_REL, EMB, NC, HALF).transpose(2, 0, 1, 3)
  W2h = W2.reshape(N_REL, HID, NC, HALF).transpose(2, 0, 1, 3)
  b1r = b1[None, :]
  b2r = b2[None, :]
  lwp = jnp.zeros((HID, 128), _f32).at[:, :N_CLASS].set(lin_W)
  lbp = jnp.zeros((1, 128), _f32).at[:, :N_CLASS].set(lin_b[None, :])

  s_edge = _sc_scales(cidx)

  r1, t1h = _t1_call(c2, p2, color_emb, pos_emb, root1, b1r, W1h)
  acc1 = _sc_agg(t1h.reshape(NC * N_REL * N_NODES, HALF), g2, d2, s_edge)
  acc1 = acc1.reshape(NC, N_NODES, HALF)

  r2, t2h = _t2_call(r1, acc1, root2, b2r, W2h)
  acc2 = _sc_agg(t2h.reshape(NC * N_REL * N_NODES, HALF), g2, d2, s_edge)
  acc2 = acc2.reshape(NC, N_NODES, HALF)

  out = _t3_call(r2, acc2, batch2, lwp, lbp)
  return out[:, :N_CLASS]


# speedup vs baseline: 11.5184x; 1.1778x over previous
import functools

import jax
import jax.numpy as jnp
from jax import lax
from jax.experimental import pallas as pl
from jax.experimental.pallas import tpu as pltpu
from jax.experimental.pallas import tpu_sc as plsc

N_NODES = 10000
N_EDGES = 320000
N_COLOR = 16
MAX_POS = 1024
EMB = 128
HID = 256
N_CLASS = 10
N_REL = 3
N_GRAPHS = 64

HALF = HID // 2
NC = 2
NS = 16
RB = 1000
GRID = N_NODES // RB
EPT = N_EDGES // NS
SCH = 400
ACH = 80
BLK = 4000
NBLK = EPT // BLK
CPB = BLK // ACH
PAIRS = CPB // 2
CROWS = EPT // ACH
ACC_PAD = 10240
ROWS_PT = ACC_PAD // NS
CNT_PAD = 30720
ZPT = CNT_PAD // NS
ZSTEP = 128

_f32 = jnp.float32
_i32 = jnp.int32


def _mm(a, b):
  return lax.dot_general(a, b, (((1,), (0,)), ((), ())),
                         precision=lax.Precision.HIGHEST,
                         preferred_element_type=_f32)


def _t1_body(c_ref, p_ref, ce_ref, pe_ref, root_ref, b_ref, w_ref,
             r1_ref, th_ref):
  c = c_ref[:, 0]
  p = p_ref[:, 0]
  ohc = (c[:, None] == lax.broadcasted_iota(_i32, (RB, N_COLOR), 1)
         ).astype(_f32)
  ohp = (p[:, None] == lax.broadcasted_iota(_i32, (RB, MAX_POS), 1)
         ).astype(_f32)
  h0 = _mm(ohc, ce_ref[...]) + _mm(ohp, pe_ref[...])
  r1_ref[...] = _mm(h0, root_ref[...]) + b_ref[...]
  for ch in range(NC):
    for r in range(N_REL):
      th_ref[ch, r] = _mm(h0, w_ref[ch, r])


_t1_call = pl.pallas_call(
    _t1_body,
    grid=(GRID,),
    in_specs=[
        pl.BlockSpec((RB, 1), lambda i: (i, 0)),
        pl.BlockSpec((RB, 1), lambda i: (i, 0)),
        pl.BlockSpec((N_COLOR, EMB), lambda i: (0, 0)),
        pl.BlockSpec((MAX_POS, EMB), lambda i: (0, 0)),
        pl.BlockSpec((EMB, HID), lambda i: (0, 0)),
        pl.BlockSpec((1, HID), lambda i: (0, 0)),
        pl.BlockSpec((NC, N_REL, EMB, HALF), lambda i: (0, 0, 0, 0)),
    ],
    out_specs=[
        pl.BlockSpec((RB, HID), lambda i: (i, 0)),
        pl.BlockSpec((NC, N_REL, RB, HALF), lambda i: (0, 0, i, 0)),
    ],
    out_shape=[
        jax.ShapeDtypeStruct((N_NODES, HID), _f32),
        jax.ShapeDtypeStruct((NC, N_REL, N_NODES, HALF), _f32),
    ],
)


def _t2_body(rin_ref, acc_ref, root_ref, b_ref, w_ref, r2_ref, th_ref):
  h = jnp.maximum(
      rin_ref[...] + jnp.concatenate([acc_ref[0], acc_ref[1]], axis=-1), 0.0)
  r2_ref[...] = _mm(h, root_ref[...]) + b_ref[...]
  for ch in range(NC):
    for r in range(N_REL):
      th_ref[ch, r] = _mm(h, w_ref[ch, r])


_t2_call = pl.pallas_call(
    _t2_body,
    grid=(GRID,),
    in_specs=[
        pl.BlockSpec((RB, HID), lambda i: (i, 0)),
        pl.BlockSpec((NC, RB, HALF), lambda i: (0, i, 0)),
        pl.BlockSpec((HID, HID), lambda i: (0, 0)),
        pl.BlockSpec((1, HID), lambda i: (0, 0)),
        pl.BlockSpec((NC, N_REL, HID, HALF), lambda i: (0, 0, 0, 0)),
    ],
    out_specs=[
        pl.BlockSpec((RB, HID), lambda i: (i, 0)),
        pl.BlockSpec((NC, N_REL, RB, HALF), lambda i: (0, 0, i, 0)),
    ],
    out_shape=[
        jax.ShapeDtypeStruct((N_NODES, HID), _f32),
        jax.ShapeDtypeStruct((NC, N_REL, N_NODES, HALF), _f32),
    ],
)


def _t3_body(rin_ref, acc_ref, b_ref, lw_ref, lb_ref, out_ref,
             pool_ref, cnt_ref):
  i = pl.program_id(0)

  @pl.when(i == 0)
  def _():
    pool_ref[...] = jnp.zeros_like(pool_ref)
    cnt_ref[...] = jnp.zeros_like(cnt_ref)

  h = jnp.maximum(
      rin_ref[...] + jnp.concatenate([acc_ref[0], acc_ref[1]], axis=-1), 0.0)
  bid = b_ref[:, 0]
  oh = (bid[:, None] == lax.broadcasted_iota(_i32, (RB, N_GRAPHS), 1)
        ).astype(_f32)
  tdot = functools.partial(lax.dot_general,
                           dimension_numbers=(((0,), (0,)), ((), ())),
                           precision=lax.Precision.HIGHEST,
                           preferred_element_type=_f32)
  pool_ref[...] += tdot(oh, h)
  cnt_ref[...] += tdot(oh, jnp.ones((RB, 128), _f32))

  @pl.when(i == GRID - 1)
  def _():
    pooled = pool_ref[...] / jnp.maximum(cnt_ref[:, :1], 1.0)
    out_ref[...] = _mm(pooled, lw_ref[...]) + lb_ref[...]


_t3_call = pl.pallas_call(
    _t3_body,
    grid=(GRID,),
    in_specs=[
        pl.BlockSpec((RB, HID), lambda i: (i, 0)),
        pl.BlockSpec((NC, RB, HALF), lambda i: (0, i, 0)),
        pl.BlockSpec((RB, 1), lambda i: (i, 0)),
        pl.BlockSpec((HID, 128), lambda i: (0, 0)),
        pl.BlockSpec((1, 128), lambda i: (0, 0)),
    ],
    out_specs=pl.BlockSpec((N_GRAPHS, 128), lambda i: (0, 0)),
    out_shape=jax.ShapeDtypeStruct((N_GRAPHS, 128), _f32),
    scratch_shapes=[
        pltpu.VMEM((N_GRAPHS, HID), _f32),
        pltpu.VMEM((N_GRAPHS, 128), _f32),
    ],
)


_sc_mesh = plsc.VectorSubcoreMesh(core_axis_name="c", subcore_axis_name="s")


@functools.partial(
    pl.kernel,
    out_type=jax.ShapeDtypeStruct((N_EDGES,), _f32),
    mesh=_sc_mesh,
    compiler_params=pltpu.CompilerParams(needs_layout_passes=False),
    scratch_types=[
        pltpu.VMEM_SHARED((CNT_PAD,), _f32),
        pltpu.VMEM((SCH,), _i32),
        pltpu.VMEM((SCH,), _f32),
        pltpu.VMEM((SCH,), _f32),
        pltpu.VMEM((CNT_PAD,), _f32),
    ],
)
def _sc_scales(cidx_hbm, s_hbm, cnt_sh, idx_v, ones_v, s_v, cnt_v):
  cid = lax.axis_index("c")
  sid = lax.axis_index("s")

  def fill(i, carry):
    ones_v[pl.ds(i * 16, 16)] = jnp.full((16,), 1.0, _f32)
    s_v[pl.ds(i * 16, 16)] = jnp.zeros((16,), _f32)
    return carry
  lax.fori_loop(0, SCH // 16, fill, 0)

  def zero(i, carry):
    pltpu.sync_copy(s_v.at[pl.ds(0, ZSTEP)],
                    cnt_sh.at[pl.ds(sid * ZPT + i * ZSTEP, ZSTEP)])
    return carry
  lax.fori_loop(0, ZPT // ZSTEP, zero, 0)
  plsc.subcore_barrier()

  def chunk_a(i, carry):
    base = sid * EPT + i * SCH
    pltpu.sync_copy(cidx_hbm.at[pl.ds(base, SCH)], idx_v)
    pltpu.sync_copy(ones_v, cnt_sh.at[idx_v], add=True)
    return carry
  lax.fori_loop(0, EPT // SCH, chunk_a, 0)
  plsc.subcore_barrier()

  pltpu.sync_copy(cnt_sh, cnt_v)
  ebase = cid * (N_EDGES // NC) + sid * (EPT // NC)

  def chunk_b(i, carry):
    b = ebase + i * SCH
    pltpu.sync_copy(cidx_hbm.at[pl.ds(b, SCH)], idx_v)

    def grp(j, c2):
      iv = idx_v[pl.ds(j * 16, 16)]
      cv = plsc.load_gather(cnt_v, [iv])
      s_v[pl.ds(j * 16, 16)] = 1.0 / jnp.maximum(cv, 1.0)
      return c2
    lax.fori_loop(0, SCH // 16, grp, 0)
    pltpu.sync_copy(s_v, s_hbm.at[pl.ds(b, SCH)])
    return carry
  lax.fori_loop(0, EPT // NC // SCH, chunk_b, 0)


@functools.partial(
    pl.kernel,
    out_type=jax.ShapeDtypeStruct((NC * N_NODES, HALF), _f32),
    mesh=_sc_mesh,
    compiler_params=pltpu.CompilerParams(needs_layout_passes=False),
    scratch_types=[
        pltpu.VMEM_SHARED((ACC_PAD, HALF), _f32),
        pltpu.VMEM((CPB, ACH), _i32),
        pltpu.VMEM((CPB, ACH), _i32),
        pltpu.VMEM((BLK,), _f32),
        pltpu.VMEM((ACH, HALF), _f32),
        pltpu.VMEM((ACH, HALF), _f32),
        pltpu.VMEM((16, HALF), _f32),
        pltpu.SemaphoreType.DMA,
        pltpu.SemaphoreType.DMA,
        pltpu.SemaphoreType.DMA,
    ],
)
def _sc_agg(th_hbm, g_hbm, dst_hbm, s_hbm, out_hbm,
            acc_sh, g2_v, d2_v, s_v, rows_a, rows_b, z_v,
            sem_ga, sem_gb, sem_sa):
  cid = lax.axis_index("c")
  sid = lax.axis_index("s")

  def zfill(i, carry):
    r = i // (HALF // 16)
    k = i % (HALF // 16)
    z_v[r, pl.ds(k * 16, 16)] = jnp.zeros((16,), _f32)
    return carry
  lax.fori_loop(0, 16 * (HALF // 16), zfill, 0)

  rbase = sid * ROWS_PT

  def zero(i, carry):
    pltpu.sync_copy(z_v, acc_sh.at[pl.ds(rbase + i * 16, 16)])
    return carry
  lax.fori_loop(0, ROWS_PT // 16, zero, 0)
  plsc.subcore_barrier()

  off = cid * (N_REL * N_NODES)

  def _scale(rows, c):
    @plsc.parallel_loop(0, ACH, step=1, unroll=4)
    def _edge(j):
      sv = plsc.load_gather(s_v, [jnp.full((16,), c * ACH + j, _i32)])
      for k in range(HALF // 16):
        rows[j, pl.ds(k * 16, 16)] = rows[j, pl.ds(k * 16, 16)] * sv

  def block(b, carry):
    blkid = sid * NBLK + b
    pltpu.sync_copy(g_hbm.at[blkid], g2_v)
    pltpu.sync_copy(dst_hbm.at[blkid], d2_v)
    pltpu.sync_copy(s_hbm.at[pl.ds(sid * EPT + b * BLK, BLK)], s_v)

    def adj(i, c2):
      r = i // (ACH // 16)
      k = i % (ACH // 16)
      g2_v[r, pl.ds(k * 16, 16)] = g2_v[r, pl.ds(k * 16, 16)] + off
      return c2
    lax.fori_loop(0, CPB * (ACH // 16), adj, 0)

    pltpu.async_copy(th_hbm.at[g2_v.at[0]], rows_a, sem_ga)

    def pair(p, c2):
      ca = 2 * p
      cb = 2 * p + 1
      pltpu.async_copy(th_hbm.at[g2_v.at[cb]], rows_b, sem_gb)
      pltpu.make_async_copy(th_hbm.at[g2_v.at[ca]], rows_a, sem_ga).wait()
      _scale(rows_a, ca)
      pltpu.async_copy(rows_a, acc_sh.at[d2_v.at[ca]], sem_sa, add=True)
      pltpu.make_async_copy(th_hbm.at[g2_v.at[cb]], rows_b, sem_gb).wait()
      _scale(rows_b, cb)
      pltpu.make_async_copy(rows_a, acc_sh.at[d2_v.at[ca]], sem_sa).wait()

      @pl.when(p < PAIRS - 1)
      def _():
        pltpu.async_copy(th_hbm.at[g2_v.at[ca + 2]], rows_a, sem_ga)

      pltpu.sync_copy(rows_b, acc_sh.at[d2_v.at[cb]], add=True)
      return c2
    lax.fori_loop(0, PAIRS, pair, 0)
    return carry
  lax.fori_loop(0, NBLK, block, 0)
  plsc.subcore_barrier()

  @pl.when(sid < NS - 1)
  def _():
    pltpu.sync_copy(acc_sh.at[pl.ds(rbase, ROWS_PT)],
                    out_hbm.at[pl.ds(cid * N_NODES + rbase, ROWS_PT)])

  @pl.when(sid == NS - 1)
  def _():
    pltpu.sync_copy(acc_sh.at[pl.ds(rbase, N_NODES - (NS - 1) * ROWS_PT)],
                    out_hbm.at[pl.ds(cid * N_NODES + rbase,
                                     N_NODES - (NS - 1) * ROWS_PT)])


def kernel(x, edge_index, edge_type, batch, color_emb, pos_emb,
           W1, root1, b1, W2, root2, b2, lin_W, lin_b):
  xi = x.astype(_i32)
  c2 = xi[:, 1:2]
  p2 = xi[:, 2:3]
  src = edge_index[0].astype(_i32)
  dst = edge_index[1].astype(_i32)
  rel = edge_type.astype(_i32)
  g2 = (rel * N_NODES + src).reshape(NS * NBLK, CPB, ACH)
  d2 = dst.reshape(NS * NBLK, CPB, ACH)
  cidx = rel * N_NODES + dst
  batch2 = batch.astype(_i32)[:, None]

  W1h = W1.reshape(N_REL, EMB, NC, HALF).transpose(2, 0, 1, 3)
  W2h = W2.reshape(N_REL, HID, NC, HALF).transpose(2, 0, 1, 3)
  b1r = b1[None, :]
  b2r = b2[None, :]
  lwp = jnp.zeros((HID, 128), _f32).at[:, :N_CLASS].set(lin_W)
  lbp = jnp.zeros((1, 128), _f32).at[:, :N_CLASS].set(lin_b[None, :])

  s_edge = _sc_scales(cidx)

  r1, t1h = _t1_call(c2, p2, color_emb, pos_emb, root1, b1r, W1h)
  acc1 = _sc_agg(t1h.reshape(NC * N_REL * N_NODES, HALF), g2, d2, s_edge)
  acc1 = acc1.reshape(NC, N_NODES, HALF)

  r2, t2h = _t2_call(r1, acc1, root2, b2r, W2h)
  acc2 = _sc_agg(t2h.reshape(NC * N_REL * N_NODES, HALF), g2, d2, s_edge)
  acc2 = acc2.reshape(NC, N_NODES, HALF)

  out = _t3_call(r2, acc2, batch2, lwp, lbp)
  return out[:, :N_CLASS]


# speedup vs baseline: 11.5211x; 1.0002x over previous
import functools

import jax
import jax.numpy as jnp
from jax import lax
from jax.experimental import pallas as pl
from jax.experimental.pallas import tpu as pltpu
from jax.experimental.pallas import tpu_sc as plsc

N_NODES = 10000
N_EDGES = 320000
N_COLOR = 16
MAX_POS = 1024
EMB = 128
HID = 256
N_CLASS = 10
N_REL = 3
N_GRAPHS = 64

HALF = HID // 2
NC = 2
NS = 16
RB = 1000
GRID = N_NODES // RB
EPT = N_EDGES // NS
SCH = 400
ACH = 80
BLK = 4000
NBLK = EPT // BLK
CPB = BLK // ACH
PAIRS = CPB // 2
CROWS = EPT // ACH
ACC_PAD = 10240
ROWS_PT = ACC_PAD // NS
CNT_PAD = 30720
ZPT = CNT_PAD // NS
ZSTEP = 128

_f32 = jnp.float32
_i32 = jnp.int32


def _mm(a, b):
  return lax.dot_general(a, b, (((1,), (0,)), ((), ())),
                         precision=lax.Precision.HIGHEST,
                         preferred_element_type=_f32)


def _t1_body(c_ref, p_ref, ce_ref, pe_ref, root_ref, b_ref, w_ref,
             r1_ref, th_ref):
  c = c_ref[:, 0]
  p = p_ref[:, 0]
  ohc = (c[:, None] == lax.broadcasted_iota(_i32, (RB, N_COLOR), 1)
         ).astype(_f32)
  ohp = (p[:, None] == lax.broadcasted_iota(_i32, (RB, MAX_POS), 1)
         ).astype(_f32)
  h0 = _mm(ohc, ce_ref[...]) + _mm(ohp, pe_ref[...])
  r1_ref[...] = _mm(h0, root_ref[...]) + b_ref[...]
  for ch in range(NC):
    for r in range(N_REL):
      th_ref[ch, r] = _mm(h0, w_ref[ch, r])


_t1_call = pl.pallas_call(
    _t1_body,
    grid=(GRID,),
    in_specs=[
        pl.BlockSpec((RB, 1), lambda i: (i, 0)),
        pl.BlockSpec((RB, 1), lambda i: (i, 0)),
        pl.BlockSpec((N_COLOR, EMB), lambda i: (0, 0)),
        pl.BlockSpec((MAX_POS, EMB), lambda i: (0, 0)),
        pl.BlockSpec((EMB, HID), lambda i: (0, 0)),
        pl.BlockSpec((1, HID), lambda i: (0, 0)),
        pl.BlockSpec((NC, N_REL, EMB, HALF), lambda i: (0, 0, 0, 0)),
    ],
    out_specs=[
        pl.BlockSpec((RB, HID), lambda i: (i, 0)),
        pl.BlockSpec((NC, N_REL, RB, HALF), lambda i: (0, 0, i, 0)),
    ],
    out_shape=[
        jax.ShapeDtypeStruct((N_NODES, HID), _f32),
        jax.ShapeDtypeStruct((NC, N_REL, N_NODES, HALF), _f32),
    ],
)


def _t2_body(rin_ref, acc_ref, root_ref, b_ref, w_ref, r2_ref, th_ref):
  h = jnp.maximum(
      rin_ref[...] + jnp.concatenate([acc_ref[0], acc_ref[1]], axis=-1), 0.0)
  r2_ref[...] = _mm(h, root_ref[...]) + b_ref[...]
  for ch in range(NC):
    for r in range(N_REL):
      th_ref[ch, r] = _mm(h, w_ref[ch, r])


_t2_call = pl.pallas_call(
    _t2_body,
    grid=(GRID,),
    in_specs=[
        pl.BlockSpec((RB, HID), lambda i: (i, 0)),
        pl.BlockSpec((NC, RB, HALF), lambda i: (0, i, 0)),
        pl.BlockSpec((HID, HID), lambda i: (0, 0)),
        pl.BlockSpec((1, HID), lambda i: (0, 0)),
        pl.BlockSpec((NC, N_REL, HID, HALF), lambda i: (0, 0, 0, 0)),
    ],
    out_specs=[
        pl.BlockSpec((RB, HID), lambda i: (i, 0)),
        pl.BlockSpec((NC, N_REL, RB, HALF), lambda i: (0, 0, i, 0)),
    ],
    out_shape=[
        jax.ShapeDtypeStruct((N_NODES, HID), _f32),
        jax.ShapeDtypeStruct((NC, N_REL, N_NODES, HALF), _f32),
    ],
)


def _t3_body(rin_ref, acc_ref, b_ref, lw_ref, lb_ref, out_ref,
             pool_ref, cnt_ref):
  i = pl.program_id(0)

  @pl.when(i == 0)
  def _():
    pool_ref[...] = jnp.zeros_like(pool_ref)
    cnt_ref[...] = jnp.zeros_like(cnt_ref)

  h = jnp.maximum(
      rin_ref[...] + jnp.concatenate([acc_ref[0], acc_ref[1]], axis=-1), 0.0)
  bid = b_ref[:, 0]
  oh = (bid[:, None] == lax.broadcasted_iota(_i32, (RB, N_GRAPHS), 1)
        ).astype(_f32)
  tdot = functools.partial(lax.dot_general,
                           dimension_numbers=(((0,), (0,)), ((), ())),
                           precision=lax.Precision.HIGHEST,
                           preferred_element_type=_f32)
  pool_ref[...] += tdot(oh, h)
  cnt_ref[...] += tdot(oh, jnp.ones((RB, 128), _f32))

  @pl.when(i == GRID - 1)
  def _():
    pooled = pool_ref[...] / jnp.maximum(cnt_ref[:, :1], 1.0)
    out_ref[...] = _mm(pooled, lw_ref[...]) + lb_ref[...]


_t3_call = pl.pallas_call(
    _t3_body,
    grid=(GRID,),
    in_specs=[
        pl.BlockSpec((RB, HID), lambda i: (i, 0)),
        pl.BlockSpec((NC, RB, HALF), lambda i: (0, i, 0)),
        pl.BlockSpec((RB, 1), lambda i: (i, 0)),
        pl.BlockSpec((HID, 128), lambda i: (0, 0)),
        pl.BlockSpec((1, 128), lambda i: (0, 0)),
    ],
    out_specs=pl.BlockSpec((N_GRAPHS, 128), lambda i: (0, 0)),
    out_shape=jax.ShapeDtypeStruct((N_GRAPHS, 128), _f32),
    scratch_shapes=[
        pltpu.VMEM((N_GRAPHS, HID), _f32),
        pltpu.VMEM((N_GRAPHS, 128), _f32),
    ],
)


_sc_mesh = plsc.VectorSubcoreMesh(core_axis_name="c", subcore_axis_name="s")


@functools.partial(
    pl.kernel,
    out_type=jax.ShapeDtypeStruct((N_EDGES,), _f32),
    mesh=_sc_mesh,
    compiler_params=pltpu.CompilerParams(needs_layout_passes=False),
    scratch_types=[
        pltpu.VMEM_SHARED((CNT_PAD,), _f32),
        pltpu.VMEM((SCH,), _i32),
        pltpu.VMEM((SCH,), _f32),
        pltpu.VMEM((SCH,), _f32),
        pltpu.VMEM((CNT_PAD,), _f32),
    ],
)
def _sc_scales(cidx_hbm, s_hbm, cnt_sh, idx_v, ones_v, s_v, cnt_v):
  cid = lax.axis_index("c")
  sid = lax.axis_index("s")

  def fill(i, carry):
    ones_v[pl.ds(i * 16, 16)] = jnp.full((16,), 1.0, _f32)
    s_v[pl.ds(i * 16, 16)] = jnp.zeros((16,), _f32)
    return carry
  lax.fori_loop(0, SCH // 16, fill, 0)

  def zero(i, carry):
    pltpu.sync_copy(s_v.at[pl.ds(0, ZSTEP)],
                    cnt_sh.at[pl.ds(sid * ZPT + i * ZSTEP, ZSTEP)])
    return carry
  lax.fori_loop(0, ZPT // ZSTEP, zero, 0)
  plsc.subcore_barrier()

  def chunk_a(i, carry):
    base = sid * EPT + i * SCH
    pltpu.sync_copy(cidx_hbm.at[pl.ds(base, SCH)], idx_v)
    pltpu.sync_copy(ones_v, cnt_sh.at[idx_v], add=True)
    return carry
  lax.fori_loop(0, EPT // SCH, chunk_a, 0)
  plsc.subcore_barrier()

  pltpu.sync_copy(cnt_sh, cnt_v)
  ebase = cid * (N_EDGES // NC) + sid * (EPT // NC)

  def chunk_b(i, carry):
    b = ebase + i * SCH
    pltpu.sync_copy(cidx_hbm.at[pl.ds(b, SCH)], idx_v)

    def grp(j, c2):
      iv = idx_v[pl.ds(j * 16, 16)]
      cv = plsc.load_gather(cnt_v, [iv])
      s_v[pl.ds(j * 16, 16)] = 1.0 / jnp.maximum(cv, 1.0)
      return c2
    lax.fori_loop(0, SCH // 16, grp, 0)
    pltpu.sync_copy(s_v, s_hbm.at[pl.ds(b, SCH)])
    return carry
  lax.fori_loop(0, EPT // NC // SCH, chunk_b, 0)


@functools.partial(
    pl.kernel,
    out_type=jax.ShapeDtypeStruct((NC * N_NODES, HALF), _f32),
    mesh=_sc_mesh,
    compiler_params=pltpu.CompilerParams(needs_layout_passes=False),
    scratch_types=[
        pltpu.VMEM_SHARED((ACC_PAD, HALF), _f32),
        pltpu.VMEM((CPB, ACH), _i32),
        pltpu.VMEM((CPB, ACH), _i32),
        pltpu.VMEM((BLK,), _f32),
        pltpu.VMEM((ACH, HALF), _f32),
        pltpu.VMEM((ACH, HALF), _f32),
        pltpu.VMEM((16, HALF), _f32),
        pltpu.SemaphoreType.DMA,
        pltpu.SemaphoreType.DMA,
        pltpu.SemaphoreType.DMA,
    ],
)
def _sc_agg(th_hbm, g_hbm, dst_hbm, s_hbm, out_hbm,
            acc_sh, g2_v, d2_v, s_v, rows_a, rows_b, z_v,
            sem_ga, sem_gb, sem_sa):
  cid = lax.axis_index("c")
  sid = lax.axis_index("s")

  def zfill(i, carry):
    r = i // (HALF // 16)
    k = i % (HALF // 16)
    z_v[r, pl.ds(k * 16, 16)] = jnp.zeros((16,), _f32)
    return carry
  lax.fori_loop(0, 16 * (HALF // 16), zfill, 0)

  rbase = sid * ROWS_PT

  def zero(i, carry):
    pltpu.sync_copy(z_v, acc_sh.at[pl.ds(rbase + i * 16, 16)])
    return carry
  lax.fori_loop(0, ROWS_PT // 16, zero, 0)
  plsc.subcore_barrier()

  off = cid * (N_REL * N_NODES)

  def _scale(rows, c):
    @plsc.parallel_loop(0, ACH, step=1, unroll=8)
    def _edge(j):
      sv = plsc.load_gather(s_v, [jnp.full((16,), c * ACH + j, _i32)])
      for k in range(HALF // 16):
        rows[j, pl.ds(k * 16, 16)] = rows[j, pl.ds(k * 16, 16)] * sv

  def block(b, carry):
    blkid = sid * NBLK + b
    pltpu.sync_copy(g_hbm.at[blkid], g2_v)
    pltpu.sync_copy(dst_hbm.at[blkid], d2_v)
    pltpu.sync_copy(s_hbm.at[pl.ds(sid * EPT + b * BLK, BLK)], s_v)

    def adj(i, c2):
      r = i // (ACH // 16)
      k = i % (ACH // 16)
      g2_v[r, pl.ds(k * 16, 16)] = g2_v[r, pl.ds(k * 16, 16)] + off
      return c2
    lax.fori_loop(0, CPB * (ACH // 16), adj, 0)

    pltpu.async_copy(th_hbm.at[g2_v.at[0]], rows_a, sem_ga)

    def pair(p, c2):
      ca = 2 * p
      cb = 2 * p + 1
      pltpu.async_copy(th_hbm.at[g2_v.at[cb]], rows_b, sem_gb)
      pltpu.make_async_copy(th_hbm.at[g2_v.at[ca]], rows_a, sem_ga).wait()
      _scale(rows_a, ca)
      pltpu.async_copy(rows_a, acc_sh.at[d2_v.at[ca]], sem_sa, add=True)
      pltpu.make_async_copy(th_hbm.at[g2_v.at[cb]], rows_b, sem_gb).wait()
      _scale(rows_b, cb)
      pltpu.make_async_copy(rows_a, acc_sh.at[d2_v.at[ca]], sem_sa).wait()

      @pl.when(p < PAIRS - 1)
      def _():
        pltpu.async_copy(th_hbm.at[g2_v.at[ca + 2]], rows_a, sem_ga)

      pltpu.sync_copy(rows_b, acc_sh.at[d2_v.at[cb]], add=True)
      return c2
    lax.fori_loop(0, PAIRS, pair, 0)
    return carry
  lax.fori_loop(0, NBLK, block, 0)
  plsc.subcore_barrier()

  @pl.when(sid < NS - 1)
  def _():
    pltpu.sync_copy(acc_sh.at[pl.ds(rbase, ROWS_PT)],
                    out_hbm.at[pl.ds(cid * N_NODES + rbase, ROWS_PT)])

  @pl.when(sid == NS - 1)
  def _():
    pltpu.sync_copy(acc_sh.at[pl.ds(rbase, N_NODES - (NS - 1) * ROWS_PT)],
                    out_hbm.at[pl.ds(cid * N_NODES + rbase,
                                     N_NODES - (NS - 1) * ROWS_PT)])


def kernel(x, edge_index, edge_type, batch, color_emb, pos_emb,
           W1, root1, b1, W2, root2, b2, lin_W, lin_b):
  xi = x.astype(_i32)
  c2 = xi[:, 1:2]
  p2 = xi[:, 2:3]
  src = edge_index[0].astype(_i32)
  dst = edge_index[1].astype(_i32)
  rel = edge_type.astype(_i32)
  g2 = (rel * N_NODES + src).reshape(NS * NBLK, CPB, ACH)
  d2 = dst.reshape(NS * NBLK, CPB, ACH)
  cidx = rel * N_NODES + dst
  batch2 = batch.astype(_i32)[:, None]

  W1h = W1.reshape(N_REL, EMB, NC, HALF).transpose(2, 0, 1, 3)
  W2h = W2.reshape(N_REL, HID, NC, HALF).transpose(2, 0, 1, 3)
  b1r = b1[None, :]
  b2r = b2[None, :]
  lwp = jnp.zeros((HID, 128), _f32).at[:, :N_CLASS].set(lin_W)
  lbp = jnp.zeros((1, 128), _f32).at[:, :N_CLASS].set(lin_b[None, :])

  s_edge = _sc_scales(cidx)

  r1, t1h = _t1_call(c2, p2, color_emb, pos_emb, root1, b1r, W1h)
  acc1 = _sc_agg(t1h.reshape(NC * N_REL * N_NODES, HALF), g2, d2, s_edge)
  acc1 = acc1.reshape(NC, N_NODES, HALF)

  r2, t2h = _t2_call(r1, acc1, root2, b2r, W2h)
  acc2 = _sc_agg(t2h.reshape(NC * N_REL * N_NODES, HALF), g2, d2, s_edge)
  acc2 = acc2.reshape(NC, N_NODES, HALF)

  out = _t3_call(r2, acc2, batch2, lwp, lbp)
  return out[:, :N_CLASS]


# speedup vs baseline: 12.9395x; 1.1231x over previous
import functools

import jax
import jax.numpy as jnp
from jax import lax
from jax.experimental import pallas as pl
from jax.experimental.pallas import tpu as pltpu
from jax.experimental.pallas import tpu_sc as plsc

N_NODES = 10000
N_EDGES = 320000
N_COLOR = 16
MAX_POS = 1024
EMB = 128
HID = 256
N_CLASS = 10
N_REL = 3
N_GRAPHS = 64

HALF = HID // 2
NC = 2
NS = 16
RB = 1000
GRID = N_NODES // RB
EPT = N_EDGES // NS
SCH = 400
ACH = 80
BLK = 4000
NBLK = EPT // BLK
CPB = BLK // ACH
PAIRS = CPB // 2
CROWS = EPT // ACH
ACC_PAD = 10240
ROWS_PT = ACC_PAD // NS
CNT_PAD = 30720
ZPT = CNT_PAD // NS
ZSTEP = 128

_f32 = jnp.float32
_i32 = jnp.int32


def _mm(a, b):
  return lax.dot_general(a, b, (((1,), (0,)), ((), ())),
                         precision=lax.Precision.HIGHEST,
                         preferred_element_type=_f32)


def _t1_body(c_ref, p_ref, ce_ref, pe_ref, root_ref, b_ref, w_ref,
             r1_ref, th_ref):
  c = c_ref[:, 0]
  p = p_ref[:, 0]
  ohc = (c[:, None] == lax.broadcasted_iota(_i32, (RB, N_COLOR), 1)
         ).astype(_f32)
  ohp = (p[:, None] == lax.broadcasted_iota(_i32, (RB, MAX_POS), 1)
         ).astype(_f32)
  h0 = _mm(ohc, ce_ref[...]) + _mm(ohp, pe_ref[...])
  r1_ref[...] = _mm(h0, root_ref[...]) + b_ref[...]
  for ch in range(NC):
    for r in range(N_REL):
      th_ref[ch, r] = _mm(h0, w_ref[ch, r])


_t1_call = pl.pallas_call(
    _t1_body,
    grid=(GRID,),
    in_specs=[
        pl.BlockSpec((RB, 1), lambda i: (i, 0)),
        pl.BlockSpec((RB, 1), lambda i: (i, 0)),
        pl.BlockSpec((N_COLOR, EMB), lambda i: (0, 0)),
        pl.BlockSpec((MAX_POS, EMB), lambda i: (0, 0)),
        pl.BlockSpec((EMB, HID), lambda i: (0, 0)),
        pl.BlockSpec((1, HID), lambda i: (0, 0)),
        pl.BlockSpec((NC, N_REL, EMB, HALF), lambda i: (0, 0, 0, 0)),
    ],
    out_specs=[
        pl.BlockSpec((RB, HID), lambda i: (i, 0)),
        pl.BlockSpec((NC, N_REL, RB, HALF), lambda i: (0, 0, i, 0)),
    ],
    out_shape=[
        jax.ShapeDtypeStruct((N_NODES, HID), _f32),
        jax.ShapeDtypeStruct((NC, N_REL, N_NODES, HALF), _f32),
    ],
)


def _t2_body(rin_ref, acc_ref, root_ref, b_ref, w_ref, r2_ref, th_ref):
  h = jnp.maximum(
      rin_ref[...] + jnp.concatenate([acc_ref[0], acc_ref[1]], axis=-1), 0.0)
  r2_ref[...] = _mm(h, root_ref[...]) + b_ref[...]
  for ch in range(NC):
    for r in range(N_REL):
      th_ref[ch, r] = _mm(h, w_ref[ch, r])


_t2_call = pl.pallas_call(
    _t2_body,
    grid=(GRID,),
    in_specs=[
        pl.BlockSpec((RB, HID), lambda i: (i, 0)),
        pl.BlockSpec((NC, RB, HALF), lambda i: (0, i, 0)),
        pl.BlockSpec((HID, HID), lambda i: (0, 0)),
        pl.BlockSpec((1, HID), lambda i: (0, 0)),
        pl.BlockSpec((NC, N_REL, HID, HALF), lambda i: (0, 0, 0, 0)),
    ],
    out_specs=[
        pl.BlockSpec((RB, HID), lambda i: (i, 0)),
        pl.BlockSpec((NC, N_REL, RB, HALF), lambda i: (0, 0, i, 0)),
    ],
    out_shape=[
        jax.ShapeDtypeStruct((N_NODES, HID), _f32),
        jax.ShapeDtypeStruct((NC, N_REL, N_NODES, HALF), _f32),
    ],
)


def _t3_body(rin_ref, acc_ref, b_ref, lw_ref, lb_ref, out_ref,
             pool_ref, cnt_ref):
  i = pl.program_id(0)

  @pl.when(i == 0)
  def _():
    pool_ref[...] = jnp.zeros_like(pool_ref)
    cnt_ref[...] = jnp.zeros_like(cnt_ref)

  h = jnp.maximum(
      rin_ref[...] + jnp.concatenate([acc_ref[0], acc_ref[1]], axis=-1), 0.0)
  bid = b_ref[:, 0]
  oh = (bid[:, None] == lax.broadcasted_iota(_i32, (RB, N_GRAPHS), 1)
        ).astype(_f32)
  tdot = functools.partial(lax.dot_general,
                           dimension_numbers=(((0,), (0,)), ((), ())),
                           precision=lax.Precision.HIGHEST,
                           preferred_element_type=_f32)
  pool_ref[...] += tdot(oh, h)
  cnt_ref[...] += tdot(oh, jnp.ones((RB, 128), _f32))

  @pl.when(i == GRID - 1)
  def _():
    pooled = pool_ref[...] / jnp.maximum(cnt_ref[:, :1], 1.0)
    out_ref[...] = _mm(pooled, lw_ref[...]) + lb_ref[...]


_t3_call = pl.pallas_call(
    _t3_body,
    grid=(GRID,),
    in_specs=[
        pl.BlockSpec((RB, HID), lambda i: (i, 0)),
        pl.BlockSpec((NC, RB, HALF), lambda i: (0, i, 0)),
        pl.BlockSpec((RB, 1), lambda i: (i, 0)),
        pl.BlockSpec((HID, 128), lambda i: (0, 0)),
        pl.BlockSpec((1, 128), lambda i: (0, 0)),
    ],
    out_specs=pl.BlockSpec((N_GRAPHS, 128), lambda i: (0, 0)),
    out_shape=jax.ShapeDtypeStruct((N_GRAPHS, 128), _f32),
    scratch_shapes=[
        pltpu.VMEM((N_GRAPHS, HID), _f32),
        pltpu.VMEM((N_GRAPHS, 128), _f32),
    ],
)


_sc_mesh = plsc.VectorSubcoreMesh(core_axis_name="c", subcore_axis_name="s")


@functools.partial(
    pl.kernel,
    out_type=jax.ShapeDtypeStruct((N_EDGES,), _f32),
    mesh=_sc_mesh,
    compiler_params=pltpu.CompilerParams(needs_layout_passes=False),
    scratch_types=[
        pltpu.VMEM_SHARED((CNT_PAD,), _f32),
        pltpu.VMEM((SCH,), _i32),
        pltpu.VMEM((SCH,), _f32),
        pltpu.VMEM((SCH,), _f32),
        pltpu.VMEM((CNT_PAD,), _f32),
    ],
)
def _sc_scales(cidx_hbm, s_hbm, cnt_sh, idx_v, ones_v, s_v, cnt_v):
  cid = lax.axis_index("c")
  sid = lax.axis_index("s")

  def fill(i, carry):
    ones_v[pl.ds(i * 16, 16)] = jnp.full((16,), 1.0, _f32)
    s_v[pl.ds(i * 16, 16)] = jnp.zeros((16,), _f32)
    return carry
  lax.fori_loop(0, SCH // 16, fill, 0)

  def zero(i, carry):
    pltpu.sync_copy(s_v.at[pl.ds(0, ZSTEP)],
                    cnt_sh.at[pl.ds(sid * ZPT + i * ZSTEP, ZSTEP)])
    return carry
  lax.fori_loop(0, ZPT // ZSTEP, zero, 0)
  plsc.subcore_barrier()

  def chunk_a(i, carry):
    base = sid * EPT + i * SCH
    pltpu.sync_copy(cidx_hbm.at[pl.ds(base, SCH)], idx_v)
    pltpu.sync_copy(ones_v, cnt_sh.at[idx_v], add=True)
    return carry
  lax.fori_loop(0, EPT // SCH, chunk_a, 0)
  plsc.subcore_barrier()

  pltpu.sync_copy(cnt_sh, cnt_v)
  ebase = cid * (N_EDGES // NC) + sid * (EPT // NC)

  def chunk_b(i, carry):
    b = ebase + i * SCH
    pltpu.sync_copy(cidx_hbm.at[pl.ds(b, SCH)], idx_v)

    def grp(j, c2):
      iv = idx_v[pl.ds(j * 16, 16)]
      cv = plsc.load_gather(cnt_v, [iv])
      s_v[pl.ds(j * 16, 16)] = 1.0 / jnp.maximum(cv, 1.0)
      return c2
    lax.fori_loop(0, SCH // 16, grp, 0)
    pltpu.sync_copy(s_v, s_hbm.at[pl.ds(b, SCH)])
    return carry
  lax.fori_loop(0, EPT // NC // SCH, chunk_b, 0)


@functools.partial(
    pl.kernel,
    out_type=jax.ShapeDtypeStruct((NC * N_NODES, HALF), _f32),
    mesh=_sc_mesh,
    compiler_params=pltpu.CompilerParams(needs_layout_passes=False),
    scratch_types=[
        pltpu.VMEM_SHARED((ACC_PAD, HALF), _f32),
        pltpu.VMEM((CPB, ACH), _i32),
        pltpu.VMEM((CPB, ACH), _i32),
        pltpu.VMEM((BLK,), _f32),
        pltpu.VMEM((ACH, HALF), _f32),
        pltpu.VMEM((ACH, HALF), _f32),
        pltpu.VMEM((ACH, HALF), _f32),
        pltpu.SemaphoreType.DMA,
        pltpu.SemaphoreType.DMA,
        pltpu.SemaphoreType.DMA,
        pltpu.SemaphoreType.DMA,
        pltpu.SemaphoreType.DMA,
        pltpu.SemaphoreType.DMA,
    ],
)
def _sc_agg(th_hbm, g_hbm, dst_hbm, s_hbm, out_hbm,
            acc_sh, g2_v, d2_v, s_v, rows_a, rows_b, rows_c,
            sem_ga, sem_gb, sem_gc, sem_sa, sem_sb, sem_sc):
  cid = lax.axis_index("c")
  sid = lax.axis_index("s")

  def zfill(i, carry):
    r = i // (HALF // 16)
    k = i % (HALF // 16)
    rows_a[r, pl.ds(k * 16, 16)] = jnp.zeros((16,), _f32)
    return carry
  lax.fori_loop(0, 16 * (HALF // 16), zfill, 0)

  rbase = sid * ROWS_PT

  def zero(i, carry):
    pltpu.sync_copy(rows_a.at[pl.ds(0, 16)], acc_sh.at[pl.ds(rbase + i * 16, 16)])
    return carry
  lax.fori_loop(0, ROWS_PT // 16, zero, 0)
  plsc.subcore_barrier()

  off = cid * (N_REL * N_NODES)

  def _scale(rows, c):
    @plsc.parallel_loop(0, ACH, step=1, unroll=8)
    def _edge(j):
      sv = plsc.load_gather(s_v, [jnp.full((16,), c * ACH + j, _i32)])
      for k in range(HALF // 16):
        rows[j, pl.ds(k * 16, 16)] = rows[j, pl.ds(k * 16, 16)] * sv

  def block(b, carry):
    blkid = sid * NBLK + b
    pltpu.sync_copy(g_hbm.at[blkid], g2_v)
    pltpu.sync_copy(dst_hbm.at[blkid], d2_v)
    pltpu.sync_copy(s_hbm.at[pl.ds(sid * EPT + b * BLK, BLK)], s_v)

    def adj(i, c2):
      r = i // (ACH // 16)
      k = i % (ACH // 16)
      g2_v[r, pl.ds(k * 16, 16)] = g2_v[r, pl.ds(k * 16, 16)] + off
      return c2
    lax.fori_loop(0, CPB * (ACH // 16), adj, 0)

    bufs = ((rows_a, sem_ga, sem_sa),
            (rows_b, sem_gb, sem_sb),
            (rows_c, sem_gc, sem_sc))

    def proc(c, off):
      rows, sg, ss = bufs[off]
      pltpu.make_async_copy(th_hbm.at[g2_v.at[c]], rows, sg).wait()
      _scale(rows, c)
      pltpu.async_copy(rows, acc_sh.at[d2_v.at[c]], ss, add=True)
      nrows, nsg, nss = bufs[(off + 2) % 3]
      nc = c + 2

      @pl.when(nc < CPB)
      def _():
        @pl.when(c >= 1)
        def _():
          pltpu.make_async_copy(nrows, acc_sh.at[d2_v.at[c - 1]], nss).wait()
        pltpu.async_copy(th_hbm.at[g2_v.at[nc]], nrows, nsg)

    pltpu.async_copy(th_hbm.at[g2_v.at[0]], rows_a, sem_ga)
    pltpu.async_copy(th_hbm.at[g2_v.at[1]], rows_b, sem_gb)

    def triple(t, c2):
      for off in range(3):
        proc(3 * t + off, off)
      return c2
    lax.fori_loop(0, CPB // 3, triple, 0)
    proc(CPB - 2, (CPB - 2) % 3)
    proc(CPB - 1, (CPB - 1) % 3)
    for cc in (CPB - 3, CPB - 2, CPB - 1):
      rows, _, ss = bufs[cc % 3]
      pltpu.make_async_copy(rows, acc_sh.at[d2_v.at[cc]], ss).wait()
    return carry
  lax.fori_loop(0, NBLK, block, 0)
  plsc.subcore_barrier()

  @pl.when(sid < NS - 1)
  def _():
    pltpu.sync_copy(acc_sh.at[pl.ds(rbase, ROWS_PT)],
                    out_hbm.at[pl.ds(cid * N_NODES + rbase, ROWS_PT)])

  @pl.when(sid == NS - 1)
  def _():
    pltpu.sync_copy(acc_sh.at[pl.ds(rbase, N_NODES - (NS - 1) * ROWS_PT)],
                    out_hbm.at[pl.ds(cid * N_NODES + rbase,
                                     N_NODES - (NS - 1) * ROWS_PT)])


def kernel(x, edge_index, edge_type, batch, color_emb, pos_emb,
           W1, root1, b1, W2, root2, b2, lin_W, lin_b):
  xi = x.astype(_i32)
  c2 = xi[:, 1:2]
  p2 = xi[:, 2:3]
  src = edge_index[0].astype(_i32)
  dst = edge_index[1].astype(_i32)
  rel = edge_type.astype(_i32)
  g2 = (rel * N_NODES + src).reshape(NS * NBLK, CPB, ACH)
  d2 = dst.reshape(NS * NBLK, CPB, ACH)
  cidx = rel * N_NODES + dst
  batch2 = batch.astype(_i32)[:, None]

  W1h = W1.reshape(N_REL, EMB, NC, HALF).transpose(2, 0, 1, 3)
  W2h = W2.reshape(N_REL, HID, NC, HALF).transpose(2, 0, 1, 3)
  b1r = b1[None, :]
  b2r = b2[None, :]
  lwp = jnp.zeros((HID, 128), _f32).at[:, :N_CLASS].set(lin_W)
  lbp = jnp.zeros((1, 128), _f32).at[:, :N_CLASS].set(lin_b[None, :])

  s_edge = _sc_scales(cidx)

  r1, t1h = _t1_call(c2, p2, color_emb, pos_emb, root1, b1r, W1h)
  acc1 = _sc_agg(t1h.reshape(NC * N_REL * N_NODES, HALF), g2, d2, s_edge)
  acc1 = acc1.reshape(NC, N_NODES, HALF)

  r2, t2h = _t2_call(r1, acc1, root2, b2r, W2h)
  acc2 = _sc_agg(t2h.reshape(NC * N_REL * N_NODES, HALF), g2, d2, s_edge)
  acc2 = acc2.reshape(NC, N_NODES, HALF)

  out = _t3_call(r2, acc2, batch2, lwp, lbp)
  return out[:, :N_CLASS]
